# Initial kernel scaffold; baseline (speedup 1.0000x reference)
import functools

import jax
import jax.numpy as jnp
from jax import lax
from jax.experimental import pallas as pl
from jax.experimental.pallas import tpu as pltpu
from jax.experimental.pallas import tpu_sc as plsc

N, E, H = 10000, 320000, 128
NC, NS = 2, 16
CH = 80
EPW = E // (NC * NS)
CPW = EPW // CH
NPT = N // NS

_MESH = plsc.VectorSubcoreMesh(core_axis_name="c", subcore_axis_name="s")


@functools.partial(
    pl.kernel,
    mesh=_MESH,
    out_type=jax.ShapeDtypeStruct((NC, N, H), jnp.float32),
    scratch_types=[
        pltpu.VMEM((CPW, CH), jnp.int32),
        pltpu.VMEM((CH, H), jnp.float32),
        pltpu.VMEM_SHARED((N, H), jnp.float32),
    ],
)
def _seg_sum_sc(e_hbm, end_hbm, zeros_hbm, out_hbm, idx_v, rows_v, acc_sh):
    c = lax.axis_index("c")
    s = lax.axis_index("s")
    pltpu.sync_copy(zeros_hbm.at[pl.ds(s * NPT, NPT)],
                    acc_sh.at[pl.ds(s * NPT, NPT)])
    pltpu.sync_copy(end_hbm.at[c, s], idx_v)
    plsc.subcore_barrier()
    wbase = (c * NS + s) * EPW

    def body(i, carry):
        pltpu.sync_copy(e_hbm.at[pl.ds(wbase + i * CH, CH)], rows_v)
        pltpu.sync_copy(rows_v, acc_sh.at[idx_v.at[i]], add=True)
        return carry

    lax.fori_loop(0, CPW, body, 0)
    plsc.subcore_barrier()
    pltpu.sync_copy(acc_sh.at[pl.ds(s * NPT, NPT)],
                    out_hbm.at[c, pl.ds(s * NPT, NPT)])


@functools.partial(
    pl.kernel,
    mesh=_MESH,
    out_type=(jax.ShapeDtypeStruct((E, H), jnp.float32),
              jax.ShapeDtypeStruct((E, H), jnp.float32)),
    scratch_types=[
        pltpu.VMEM((CPW, CH), jnp.int32),
        pltpu.VMEM((CPW, CH), jnp.int32),
        pltpu.VMEM((CH, H), jnp.float32),
        pltpu.VMEM((CH, H), jnp.float32),
        pltpu.SemaphoreType.DMA,
        pltpu.SemaphoreType.DMA,
    ],
)
def _gather_sc(ga_hbm, gb_hbm, start_hbm, end_hbm, outa_hbm, outb_hbm,
               idxs_v, idxe_v, bufa_v, bufb_v, sema, semb):
    c = lax.axis_index("c")
    s = lax.axis_index("s")
    pltpu.sync_copy(start_hbm.at[c, s], idxs_v)
    pltpu.sync_copy(end_hbm.at[c, s], idxe_v)
    wbase = (c * NS + s) * EPW

    def body(i, carry):
        da = pltpu.async_copy(ga_hbm.at[idxs_v.at[i]], bufa_v, sema)
        db = pltpu.async_copy(gb_hbm.at[idxe_v.at[i]], bufb_v, semb)
        da.wait()
        db.wait()
        base = wbase + i * CH
        pltpu.sync_copy(bufa_v, outa_hbm.at[pl.ds(base, CH)])
        pltpu.sync_copy(bufb_v, outb_hbm.at[pl.ds(base, CH)])
        return carry

    lax.fori_loop(0, CPW, body, 0)


def _node_body(x_ref, m_ref, w1a_ref, w1b_ref, b1_ref, w2_ref, b2_ref,
               wa_ref, wb_ref, be_ref, xout_ref, ga_ref, gb_ref):
    xb = x_ref[...]
    m = m_ref[0] + m_ref[1]
    h = jnp.maximum(
        jnp.dot(xb, w1a_ref[...], preferred_element_type=jnp.float32)
        + jnp.dot(m, w1b_ref[...], preferred_element_type=jnp.float32)
        + b1_ref[...], 0.0)
    xo = jnp.dot(h, w2_ref[...], preferred_element_type=jnp.float32) \
        + b2_ref[...] + xb
    xout_ref[...] = xo
    ga_ref[...] = jnp.dot(xo, wa_ref[...],
                          preferred_element_type=jnp.float32) + be_ref[...]
    gb_ref[...] = jnp.dot(xo, wb_ref[...], preferred_element_type=jnp.float32)


def _node_tc(x, msgs, w1a, w1b, b1n, w2n, b2n, wea, web, b1e):
    BN = 1000
    grid = (N // BN,)
    row_spec = pl.BlockSpec((BN, H), lambda i: (i, 0))
    w_spec = pl.BlockSpec((H, H), lambda i: (0, 0))
    b_spec = pl.BlockSpec((1, H), lambda i: (0, 0))
    return pl.pallas_call(
        _node_body,
        grid=grid,
        in_specs=[
            row_spec,
            pl.BlockSpec((NC, BN, H), lambda i: (0, i, 0)),
            w_spec, w_spec, b_spec, w_spec, b_spec, w_spec, w_spec, b_spec,
        ],
        out_specs=[row_spec, row_spec, row_spec],
        out_shape=[jax.ShapeDtypeStruct((N, H), jnp.float32)] * 3,
    )(x, msgs, w1a, w1b, b1n, w2n, b2n, wea, web, b1e)


def _edge_body(ga_ref, gb_ref, e_ref, wc_ref, w2_ref, b2_ref, out_ref):
    eb = e_ref[...]
    h = jnp.maximum(
        ga_ref[...] + gb_ref[...]
        + jnp.dot(eb, wc_ref[...], preferred_element_type=jnp.float32), 0.0)
    out_ref[...] = jnp.dot(h, w2_ref[...],
                           preferred_element_type=jnp.float32) \
        + b2_ref[...] + eb


def _edge_tc(ga, gb, e, wec, w2e, b2e):
    BE = 2000
    grid = (E // BE,)
    row_spec = pl.BlockSpec((BE, H), lambda i: (i, 0))
    return pl.pallas_call(
        _edge_body,
        grid=grid,
        in_specs=[
            row_spec, row_spec, row_spec,
            pl.BlockSpec((H, H), lambda i: (0, 0)),
            pl.BlockSpec((H, H), lambda i: (0, 0)),
            pl.BlockSpec((1, H), lambda i: (0, 0)),
        ],
        out_specs=row_spec,
        out_shape=jax.ShapeDtypeStruct((E, H), jnp.float32),
    )(ga, gb, e, wec, w2e, b2e)


def kernel(x, edge_index, e, W1n, b1n, W2n, b2n, W1e, b1e, W2e, b2e):
    start = edge_index[0].reshape(NC, NS, CPW, CH)
    end = edge_index[1].reshape(NC, NS, CPW, CH)
    zeros = jnp.zeros((N, H), jnp.float32)

    msgs = _seg_sum_sc(e, end, zeros)

    x_out, ga_nodes, gb_nodes = _node_tc(
        x, msgs,
        W1n[:H], W1n[H:], b1n.reshape(1, H), W2n, b2n.reshape(1, H),
        W1e[:H], W1e[H:2 * H], b1e.reshape(1, H))

    ga, gb = _gather_sc(ga_nodes, gb_nodes, start, end)

    e_out = _edge_tc(ga, gb, e, W1e[2 * H:], W2e, b2e.reshape(1, H))
    return (x_out, e_out)


# speedup vs baseline: 3.2678x; 3.2678x over previous
import functools

import jax
import jax.numpy as jnp
from jax import lax
from jax.experimental import pallas as pl
from jax.experimental.pallas import tpu as pltpu
from jax.experimental.pallas import tpu_sc as plsc

N, E, H = 10000, 320000, 128
NC, NS = 2, 16
CH = 80
EPW = E // (NC * NS)
CPW = EPW // CH
NPT = 624
NREM = N - NS * NPT

_MESH = plsc.VectorSubcoreMesh(core_axis_name="c", subcore_axis_name="s")


@functools.partial(
    pl.kernel,
    mesh=_MESH,
    out_type=jax.ShapeDtypeStruct((NC, N, H), jnp.float32),
    scratch_types=[
        pltpu.VMEM((CPW, CH), jnp.int32),
        pltpu.VMEM((CH, H), jnp.float32),
        pltpu.VMEM_SHARED((N, H), jnp.float32),
    ],
)
def _seg_sum_sc(e_hbm, end_hbm, zeros_hbm, out_hbm, idx_v, rows_v, acc_sh):
    c = lax.axis_index("c")
    s = lax.axis_index("s")
    pltpu.sync_copy(zeros_hbm.at[pl.ds(s * NPT, NPT)],
                    acc_sh.at[pl.ds(s * NPT, NPT)])

    @pl.when(s == NS - 1)
    def _zero_tail():
        pltpu.sync_copy(zeros_hbm.at[pl.ds(NS * NPT, NREM)],
                        acc_sh.at[pl.ds(NS * NPT, NREM)])

    pltpu.sync_copy(end_hbm.at[c, s], idx_v)
    plsc.subcore_barrier()
    wbase = (c * NS + s) * EPW

    def body(i, carry):
        pltpu.sync_copy(e_hbm.at[pl.ds(wbase + i * CH, CH)], rows_v)
        pltpu.sync_copy(rows_v, acc_sh.at[idx_v.at[i]], add=True)
        return carry

    lax.fori_loop(0, CPW, body, 0)
    plsc.subcore_barrier()
    pltpu.sync_copy(acc_sh.at[pl.ds(s * NPT, NPT)],
                    out_hbm.at[c, pl.ds(s * NPT, NPT)])

    @pl.when(s == NS - 1)
    def _write_tail():
        pltpu.sync_copy(acc_sh.at[pl.ds(NS * NPT, NREM)],
                        out_hbm.at[c, pl.ds(NS * NPT, NREM)])


@functools.partial(
    pl.kernel,
    mesh=_MESH,
    out_type=(jax.ShapeDtypeStruct((E, H), jnp.float32),
              jax.ShapeDtypeStruct((E, H), jnp.float32)),
    scratch_types=[
        pltpu.VMEM((CPW, CH), jnp.int32),
        pltpu.VMEM((CPW, CH), jnp.int32),
        pltpu.VMEM((CH, H), jnp.float32),
        pltpu.VMEM((CH, H), jnp.float32),
        pltpu.SemaphoreType.DMA,
        pltpu.SemaphoreType.DMA,
    ],
)
def _gather_sc(ga_hbm, gb_hbm, start_hbm, end_hbm, outa_hbm, outb_hbm,
               idxs_v, idxe_v, bufa_v, bufb_v, sema, semb):
    c = lax.axis_index("c")
    s = lax.axis_index("s")
    pltpu.sync_copy(start_hbm.at[c, s], idxs_v)
    pltpu.sync_copy(end_hbm.at[c, s], idxe_v)
    wbase = (c * NS + s) * EPW

    def body(i, carry):
        da = pltpu.async_copy(ga_hbm.at[idxs_v.at[i]], bufa_v, sema)
        db = pltpu.async_copy(gb_hbm.at[idxe_v.at[i]], bufb_v, semb)
        da.wait()
        db.wait()
        base = wbase + i * CH
        pltpu.sync_copy(bufa_v, outa_hbm.at[pl.ds(base, CH)])
        pltpu.sync_copy(bufb_v, outb_hbm.at[pl.ds(base, CH)])
        return carry

    lax.fori_loop(0, CPW, body, 0)


def _node_body(x_ref, m_ref, w1a_ref, w1b_ref, b1_ref, w2_ref, b2_ref,
               wa_ref, wb_ref, be_ref, xout_ref, ga_ref, gb_ref):
    xb = x_ref[...]
    m = m_ref[0] + m_ref[1]
    h = jnp.maximum(
        jnp.dot(xb, w1a_ref[...], preferred_element_type=jnp.float32)
        + jnp.dot(m, w1b_ref[...], preferred_element_type=jnp.float32)
        + b1_ref[...], 0.0)
    xo = jnp.dot(h, w2_ref[...], preferred_element_type=jnp.float32) \
        + b2_ref[...] + xb
    xout_ref[...] = xo
    ga_ref[...] = jnp.dot(xo, wa_ref[...],
                          preferred_element_type=jnp.float32) + be_ref[...]
    gb_ref[...] = jnp.dot(xo, wb_ref[...], preferred_element_type=jnp.float32)


def _node_tc(x, msgs, w1a, w1b, b1n, w2n, b2n, wea, web, b1e):
    BN = 1000
    grid = (N // BN,)
    row_spec = pl.BlockSpec((BN, H), lambda i: (i, 0))
    w_spec = pl.BlockSpec((H, H), lambda i: (0, 0))
    b_spec = pl.BlockSpec((1, H), lambda i: (0, 0))
    return pl.pallas_call(
        _node_body,
        grid=grid,
        in_specs=[
            row_spec,
            pl.BlockSpec((NC, BN, H), lambda i: (0, i, 0)),
            w_spec, w_spec, b_spec, w_spec, b_spec, w_spec, w_spec, b_spec,
        ],
        out_specs=[row_spec, row_spec, row_spec],
        out_shape=[jax.ShapeDtypeStruct((N, H), jnp.float32)] * 3,
    )(x, msgs, w1a, w1b, b1n, w2n, b2n, wea, web, b1e)


def _edge_body(ga_ref, gb_ref, e_ref, wc_ref, w2_ref, b2_ref, out_ref):
    eb = e_ref[...]
    h = jnp.maximum(
        ga_ref[...] + gb_ref[...]
        + jnp.dot(eb, wc_ref[...], preferred_element_type=jnp.float32), 0.0)
    out_ref[...] = jnp.dot(h, w2_ref[...],
                           preferred_element_type=jnp.float32) \
        + b2_ref[...] + eb


def _edge_tc(ga, gb, e, wec, w2e, b2e):
    BE = 2000
    grid = (E // BE,)
    row_spec = pl.BlockSpec((BE, H), lambda i: (i, 0))
    return pl.pallas_call(
        _edge_body,
        grid=grid,
        in_specs=[
            row_spec, row_spec, row_spec,
            pl.BlockSpec((H, H), lambda i: (0, 0)),
            pl.BlockSpec((H, H), lambda i: (0, 0)),
            pl.BlockSpec((1, H), lambda i: (0, 0)),
        ],
        out_specs=row_spec,
        out_shape=jax.ShapeDtypeStruct((E, H), jnp.float32),
    )(ga, gb, e, wec, w2e, b2e)


def kernel(x, edge_index, e, W1n, b1n, W2n, b2n, W1e, b1e, W2e, b2e):
    start = edge_index[0].reshape(NC, NS, CPW, CH)
    end = edge_index[1].reshape(NC, NS, CPW, CH)
    zeros = jnp.zeros((N, H), jnp.float32)

    msgs = _seg_sum_sc(e, end, zeros)

    x_out, ga_nodes, gb_nodes = _node_tc(
        x, msgs,
        W1n[:H], W1n[H:], b1n.reshape(1, H), W2n, b2n.reshape(1, H),
        W1e[:H], W1e[H:2 * H], b1e.reshape(1, H))

    ga, gb = _gather_sc(ga_nodes, gb_nodes, start, end)

    e_out = _edge_tc(ga, gb, e, W1e[2 * H:], W2e, b2e.reshape(1, H))
    return (x_out, e_out)


# speedup vs baseline: 4.1159x; 1.2595x over previous
import functools

import jax
import jax.numpy as jnp
from jax import lax
from jax.experimental import pallas as pl
from jax.experimental.pallas import tpu as pltpu
from jax.experimental.pallas import tpu_sc as plsc

N, E, H = 10000, 320000, 128
NC, NS = 2, 16
CH = 80
EPW = E // (NC * NS)
CPW = EPW // CH
NPT = 624
NREM = N - NS * NPT

_MESH = plsc.VectorSubcoreMesh(core_axis_name="c", subcore_axis_name="s")


@functools.partial(
    pl.kernel,
    mesh=_MESH,
    out_type=jax.ShapeDtypeStruct((NC, N, H), jnp.float32),
    scratch_types=[
        pltpu.VMEM((CPW, CH), jnp.int32),
        pltpu.VMEM((CH, H), jnp.float32),
        pltpu.VMEM((CH, H), jnp.float32),
        pltpu.VMEM_SHARED((N, H), jnp.float32),
        pltpu.SemaphoreType.DMA,
        pltpu.SemaphoreType.DMA,
    ],
)
def _seg_sum_sc(e_hbm, end_hbm, zeros_hbm, out_hbm, idx_v, rows0_v, rows1_v,
                acc_sh, sem0, sem1):
    c = lax.axis_index("c")
    s = lax.axis_index("s")
    pltpu.sync_copy(zeros_hbm.at[pl.ds(s * NPT, NPT)],
                    acc_sh.at[pl.ds(s * NPT, NPT)])

    @pl.when(s == NS - 1)
    def _zero_tail():
        pltpu.sync_copy(zeros_hbm.at[pl.ds(NS * NPT, NREM)],
                        acc_sh.at[pl.ds(NS * NPT, NREM)])

    pltpu.sync_copy(end_hbm.at[c, s], idx_v)
    plsc.subcore_barrier()
    wbase = (c * NS + s) * EPW

    def _start_in(i, rows, sem):
        pltpu.async_copy(e_hbm.at[pl.ds(wbase + i * CH, CH)], rows, sem)

    def _finish(i, rows, sem):
        pltpu.make_async_copy(e_hbm.at[pl.ds(wbase + i * CH, CH)],
                              rows, sem).wait()
        pltpu.sync_copy(rows, acc_sh.at[idx_v.at[i]], add=True)

    _start_in(0, rows0_v, sem0)

    def body(i, carry):
        b = lax.rem(i, 2)

        @pl.when(i + 1 < CPW)
        def _prefetch():
            @pl.when(b == 0)
            def _():
                _start_in(i + 1, rows1_v, sem1)

            @pl.when(b == 1)
            def _():
                _start_in(i + 1, rows0_v, sem0)

        @pl.when(b == 0)
        def _():
            _finish(i, rows0_v, sem0)

        @pl.when(b == 1)
        def _():
            _finish(i, rows1_v, sem1)

        return carry

    lax.fori_loop(0, CPW, body, 0)
    plsc.subcore_barrier()
    pltpu.sync_copy(acc_sh.at[pl.ds(s * NPT, NPT)],
                    out_hbm.at[c, pl.ds(s * NPT, NPT)])

    @pl.when(s == NS - 1)
    def _write_tail():
        pltpu.sync_copy(acc_sh.at[pl.ds(NS * NPT, NREM)],
                        out_hbm.at[c, pl.ds(NS * NPT, NREM)])


@functools.partial(
    pl.kernel,
    mesh=_MESH,
    out_type=(jax.ShapeDtypeStruct((E, H), jnp.float32),
              jax.ShapeDtypeStruct((E, H), jnp.float32)),
    scratch_types=[
        pltpu.VMEM((CPW, CH), jnp.int32),
        pltpu.VMEM((CPW, CH), jnp.int32),
        pltpu.VMEM((CH, H), jnp.float32),
        pltpu.VMEM((CH, H), jnp.float32),
        pltpu.VMEM((CH, H), jnp.float32),
        pltpu.VMEM((CH, H), jnp.float32),
        pltpu.SemaphoreType.DMA,
        pltpu.SemaphoreType.DMA,
        pltpu.SemaphoreType.DMA,
        pltpu.SemaphoreType.DMA,
    ],
)
def _gather_sc(ga_hbm, gb_hbm, start_hbm, end_hbm, outa_hbm, outb_hbm,
               idxs_v, idxe_v, bufa0_v, bufb0_v, bufa1_v, bufb1_v,
               sa0, sb0, sa1, sb1):
    c = lax.axis_index("c")
    s = lax.axis_index("s")
    pltpu.sync_copy(start_hbm.at[c, s], idxs_v)
    pltpu.sync_copy(end_hbm.at[c, s], idxe_v)
    wbase = (c * NS + s) * EPW

    def _start_in(i, ba, bb, sa, sb):
        pltpu.async_copy(ga_hbm.at[idxs_v.at[i]], ba, sa)
        pltpu.async_copy(gb_hbm.at[idxe_v.at[i]], bb, sb)

    def _finish(i, ba, bb, sa, sb):
        pltpu.make_async_copy(ga_hbm.at[idxs_v.at[i]], ba, sa).wait()
        pltpu.make_async_copy(gb_hbm.at[idxe_v.at[i]], bb, sb).wait()
        base = wbase + i * CH
        pltpu.sync_copy(ba, outa_hbm.at[pl.ds(base, CH)])
        pltpu.sync_copy(bb, outb_hbm.at[pl.ds(base, CH)])

    _start_in(0, bufa0_v, bufb0_v, sa0, sb0)

    def body(i, carry):
        b = lax.rem(i, 2)

        @pl.when(i + 1 < CPW)
        def _prefetch():
            @pl.when(b == 0)
            def _():
                _start_in(i + 1, bufa1_v, bufb1_v, sa1, sb1)

            @pl.when(b == 1)
            def _():
                _start_in(i + 1, bufa0_v, bufb0_v, sa0, sb0)

        @pl.when(b == 0)
        def _():
            _finish(i, bufa0_v, bufb0_v, sa0, sb0)

        @pl.when(b == 1)
        def _():
            _finish(i, bufa1_v, bufb1_v, sa1, sb1)

        return carry

    lax.fori_loop(0, CPW, body, 0)


def _node_body(x_ref, m_ref, w1a_ref, w1b_ref, b1_ref, w2_ref, b2_ref,
               wa_ref, wb_ref, be_ref, xout_ref, ga_ref, gb_ref):
    xb = x_ref[...]
    m = m_ref[0] + m_ref[1]
    h = jnp.maximum(
        jnp.dot(xb, w1a_ref[...], preferred_element_type=jnp.float32)
        + jnp.dot(m, w1b_ref[...], preferred_element_type=jnp.float32)
        + b1_ref[...], 0.0)
    xo = jnp.dot(h, w2_ref[...], preferred_element_type=jnp.float32) \
        + b2_ref[...] + xb
    xout_ref[...] = xo
    ga_ref[...] = jnp.dot(xo, wa_ref[...],
                          preferred_element_type=jnp.float32) + be_ref[...]
    gb_ref[...] = jnp.dot(xo, wb_ref[...], preferred_element_type=jnp.float32)


def _node_tc(x, msgs, w1a, w1b, b1n, w2n, b2n, wea, web, b1e):
    BN = 1000
    grid = (N // BN,)
    row_spec = pl.BlockSpec((BN, H), lambda i: (i, 0))
    w_spec = pl.BlockSpec((H, H), lambda i: (0, 0))
    b_spec = pl.BlockSpec((1, H), lambda i: (0, 0))
    return pl.pallas_call(
        _node_body,
        grid=grid,
        in_specs=[
            row_spec,
            pl.BlockSpec((NC, BN, H), lambda i: (0, i, 0)),
            w_spec, w_spec, b_spec, w_spec, b_spec, w_spec, w_spec, b_spec,
        ],
        out_specs=[row_spec, row_spec, row_spec],
        out_shape=[jax.ShapeDtypeStruct((N, H), jnp.float32)] * 3,
    )(x, msgs, w1a, w1b, b1n, w2n, b2n, wea, web, b1e)


def _edge_body(ga_ref, gb_ref, e_ref, wc_ref, w2_ref, b2_ref, out_ref):
    eb = e_ref[...]
    h = jnp.maximum(
        ga_ref[...] + gb_ref[...]
        + jnp.dot(eb, wc_ref[...], preferred_element_type=jnp.float32), 0.0)
    out_ref[...] = jnp.dot(h, w2_ref[...],
                           preferred_element_type=jnp.float32) \
        + b2_ref[...] + eb


def _edge_tc(ga, gb, e, wec, w2e, b2e):
    BE = 2000
    grid = (E // BE,)
    row_spec = pl.BlockSpec((BE, H), lambda i: (i, 0))
    return pl.pallas_call(
        _edge_body,
        grid=grid,
        in_specs=[
            row_spec, row_spec, row_spec,
            pl.BlockSpec((H, H), lambda i: (0, 0)),
            pl.BlockSpec((H, H), lambda i: (0, 0)),
            pl.BlockSpec((1, H), lambda i: (0, 0)),
        ],
        out_specs=row_spec,
        out_shape=jax.ShapeDtypeStruct((E, H), jnp.float32),
    )(ga, gb, e, wec, w2e, b2e)


def kernel(x, edge_index, e, W1n, b1n, W2n, b2n, W1e, b1e, W2e, b2e):
    start = edge_index[0].reshape(NC, NS, CPW, CH)
    end = edge_index[1].reshape(NC, NS, CPW, CH)
    zeros = jnp.zeros((N, H), jnp.float32)

    msgs = _seg_sum_sc(e, end, zeros)

    x_out, ga_nodes, gb_nodes = _node_tc(
        x, msgs,
        W1n[:H], W1n[H:], b1n.reshape(1, H), W2n, b2n.reshape(1, H),
        W1e[:H], W1e[H:2 * H], b1e.reshape(1, H))

    ga, gb = _gather_sc(ga_nodes, gb_nodes, start, end)

    e_out = _edge_tc(ga, gb, e, W1e[2 * H:], W2e, b2e.reshape(1, H))
    return (x_out, e_out)


# speedup vs baseline: 4.5765x; 1.1119x over previous
import functools

import jax
import jax.numpy as jnp
from jax import lax
from jax.experimental import pallas as pl
from jax.experimental.pallas import tpu as pltpu
from jax.experimental.pallas import tpu_sc as plsc

N, E, H = 10000, 320000, 128
NC, NS = 2, 16
CH = 80
EPW = E // (NC * NS)
CPW = EPW // CH
NPT = 624
NREM = N - NS * NPT

_MESH = plsc.VectorSubcoreMesh(core_axis_name="c", subcore_axis_name="s")


@functools.partial(
    pl.kernel,
    mesh=_MESH,
    out_type=jax.ShapeDtypeStruct((NC, N, H), jnp.float32),
    scratch_types=[
        pltpu.VMEM((CPW, CH), jnp.int32),
        pltpu.VMEM((CH, H), jnp.float32),
        pltpu.VMEM((CH, H), jnp.float32),
        pltpu.VMEM_SHARED((N, H), jnp.float32),
        pltpu.SemaphoreType.DMA,
        pltpu.SemaphoreType.DMA,
    ],
)
def _seg_sum_sc(e_hbm, end_hbm, zeros_hbm, out_hbm, idx_v, rows0_v, rows1_v,
                acc_sh, sem0, sem1):
    c = lax.axis_index("c")
    s = lax.axis_index("s")
    pltpu.sync_copy(zeros_hbm.at[pl.ds(s * NPT, NPT)],
                    acc_sh.at[pl.ds(s * NPT, NPT)])

    @pl.when(s == NS - 1)
    def _zero_tail():
        pltpu.sync_copy(zeros_hbm.at[pl.ds(NS * NPT, NREM)],
                        acc_sh.at[pl.ds(NS * NPT, NREM)])

    pltpu.sync_copy(end_hbm.at[c, s], idx_v)
    plsc.subcore_barrier()
    wbase = (c * NS + s) * EPW

    def _start_in(i, rows, sem):
        pltpu.async_copy(e_hbm.at[pl.ds(wbase + i * CH, CH)], rows, sem)

    def _finish(i, rows, sem):
        pltpu.make_async_copy(e_hbm.at[pl.ds(wbase + i * CH, CH)],
                              rows, sem).wait()
        pltpu.sync_copy(rows, acc_sh.at[idx_v.at[i]], add=True)

    _start_in(0, rows0_v, sem0)

    def body(i, carry):
        b = lax.rem(i, 2)

        @pl.when(i + 1 < CPW)
        def _prefetch():
            @pl.when(b == 0)
            def _():
                _start_in(i + 1, rows1_v, sem1)

            @pl.when(b == 1)
            def _():
                _start_in(i + 1, rows0_v, sem0)

        @pl.when(b == 0)
        def _():
            _finish(i, rows0_v, sem0)

        @pl.when(b == 1)
        def _():
            _finish(i, rows1_v, sem1)

        return carry

    lax.fori_loop(0, CPW, body, 0)
    plsc.subcore_barrier()
    pltpu.sync_copy(acc_sh.at[pl.ds(s * NPT, NPT)],
                    out_hbm.at[c, pl.ds(s * NPT, NPT)])

    @pl.when(s == NS - 1)
    def _write_tail():
        pltpu.sync_copy(acc_sh.at[pl.ds(NS * NPT, NREM)],
                        out_hbm.at[c, pl.ds(NS * NPT, NREM)])


@functools.partial(
    pl.kernel,
    mesh=_MESH,
    out_type=jax.ShapeDtypeStruct((E, H), jnp.float32),
    scratch_types=[
        pltpu.VMEM((CPW, CH), jnp.int32),
        pltpu.VMEM((CPW, CH), jnp.int32),
        pltpu.VMEM((CH, H), jnp.float32),
        pltpu.VMEM((CH, H), jnp.float32),
        pltpu.VMEM((CH, H), jnp.float32),
        pltpu.VMEM((CH, H), jnp.float32),
        pltpu.SemaphoreType.DMA,
        pltpu.SemaphoreType.DMA,
        pltpu.SemaphoreType.DMA,
        pltpu.SemaphoreType.DMA,
    ],
)
def _gather_sc(ga_hbm, gb_hbm, start_hbm, end_hbm, out_hbm,
               idxs_v, idxe_v, bufa0_v, bufb0_v, bufa1_v, bufb1_v,
               sa0, sb0, sa1, sb1):
    c = lax.axis_index("c")
    s = lax.axis_index("s")
    pltpu.sync_copy(start_hbm.at[c, s], idxs_v)
    pltpu.sync_copy(end_hbm.at[c, s], idxe_v)
    wbase = (c * NS + s) * EPW

    def _start_in(i, ba, bb, sa, sb):
        pltpu.async_copy(ga_hbm.at[idxs_v.at[i]], ba, sa)
        pltpu.async_copy(gb_hbm.at[idxe_v.at[i]], bb, sb)

    def _finish(i, ba, bb, sa, sb):
        pltpu.make_async_copy(ga_hbm.at[idxs_v.at[i]], ba, sa).wait()
        pltpu.make_async_copy(gb_hbm.at[idxe_v.at[i]], bb, sb).wait()

        def add_row(r, carry):
            for k in range(H // 16):
                sl = pl.ds(k * 16, 16)
                ba[r, sl] = ba[r, sl] + bb[r, sl]
            return carry

        lax.fori_loop(0, CH, add_row, 0)
        pltpu.sync_copy(ba, out_hbm.at[pl.ds(wbase + i * CH, CH)])

    _start_in(0, bufa0_v, bufb0_v, sa0, sb0)

    def body(i, carry):
        b = lax.rem(i, 2)

        @pl.when(i + 1 < CPW)
        def _prefetch():
            @pl.when(b == 0)
            def _():
                _start_in(i + 1, bufa1_v, bufb1_v, sa1, sb1)

            @pl.when(b == 1)
            def _():
                _start_in(i + 1, bufa0_v, bufb0_v, sa0, sb0)

        @pl.when(b == 0)
        def _():
            _finish(i, bufa0_v, bufb0_v, sa0, sb0)

        @pl.when(b == 1)
        def _():
            _finish(i, bufa1_v, bufb1_v, sa1, sb1)

        return carry

    lax.fori_loop(0, CPW, body, 0)


def _node_body(x_ref, m_ref, w1a_ref, w1b_ref, b1_ref, w2_ref, b2_ref,
               wa_ref, wb_ref, be_ref, xout_ref, ga_ref, gb_ref):
    xb = x_ref[...]
    m = m_ref[0] + m_ref[1]
    h = jnp.maximum(
        jnp.dot(xb, w1a_ref[...], preferred_element_type=jnp.float32)
        + jnp.dot(m, w1b_ref[...], preferred_element_type=jnp.float32)
        + b1_ref[...], 0.0)
    xo = jnp.dot(h, w2_ref[...], preferred_element_type=jnp.float32) \
        + b2_ref[...] + xb
    xout_ref[...] = xo
    ga_ref[...] = jnp.dot(xo, wa_ref[...],
                          preferred_element_type=jnp.float32) + be_ref[...]
    gb_ref[...] = jnp.dot(xo, wb_ref[...], preferred_element_type=jnp.float32)


def _node_tc(x, msgs, w1a, w1b, b1n, w2n, b2n, wea, web, b1e):
    BN = 1000
    grid = (N // BN,)
    row_spec = pl.BlockSpec((BN, H), lambda i: (i, 0))
    w_spec = pl.BlockSpec((H, H), lambda i: (0, 0))
    b_spec = pl.BlockSpec((1, H), lambda i: (0, 0))
    return pl.pallas_call(
        _node_body,
        grid=grid,
        in_specs=[
            row_spec,
            pl.BlockSpec((NC, BN, H), lambda i: (0, i, 0)),
            w_spec, w_spec, b_spec, w_spec, b_spec, w_spec, w_spec, b_spec,
        ],
        out_specs=[row_spec, row_spec, row_spec],
        out_shape=[jax.ShapeDtypeStruct((N, H), jnp.float32)] * 3,
    )(x, msgs, w1a, w1b, b1n, w2n, b2n, wea, web, b1e)


def _edge_body(gsum_ref, e_ref, wc_ref, w2_ref, b2_ref, out_ref):
    eb = e_ref[...]
    h = jnp.maximum(
        gsum_ref[...]
        + jnp.dot(eb, wc_ref[...], preferred_element_type=jnp.float32), 0.0)
    out_ref[...] = jnp.dot(h, w2_ref[...],
                           preferred_element_type=jnp.float32) \
        + b2_ref[...] + eb


def _edge_tc(gsum, e, wec, w2e, b2e):
    BE = 2000
    grid = (E // BE,)
    row_spec = pl.BlockSpec((BE, H), lambda i: (i, 0))
    return pl.pallas_call(
        _edge_body,
        grid=grid,
        in_specs=[
            row_spec, row_spec,
            pl.BlockSpec((H, H), lambda i: (0, 0)),
            pl.BlockSpec((H, H), lambda i: (0, 0)),
            pl.BlockSpec((1, H), lambda i: (0, 0)),
        ],
        out_specs=row_spec,
        out_shape=jax.ShapeDtypeStruct((E, H), jnp.float32),
    )(gsum, e, wec, w2e, b2e)


def kernel(x, edge_index, e, W1n, b1n, W2n, b2n, W1e, b1e, W2e, b2e):
    start = edge_index[0].reshape(NC, NS, CPW, CH)
    end = edge_index[1].reshape(NC, NS, CPW, CH)
    zeros = jnp.zeros((N, H), jnp.float32)

    msgs = _seg_sum_sc(e, end, zeros)

    x_out, ga_nodes, gb_nodes = _node_tc(
        x, msgs,
        W1n[:H], W1n[H:], b1n.reshape(1, H), W2n, b2n.reshape(1, H),
        W1e[:H], W1e[H:2 * H], b1e.reshape(1, H))

    gsum = _gather_sc(ga_nodes, gb_nodes, start, end)

    e_out = _edge_tc(gsum, e, W1e[2 * H:], W2e, b2e.reshape(1, H))
    return (x_out, e_out)


# speedup vs baseline: 4.9849x; 1.0892x over previous
import functools

import jax
import jax.numpy as jnp
from jax import lax
from jax.experimental import pallas as pl
from jax.experimental.pallas import tpu as pltpu
from jax.experimental.pallas import tpu_sc as plsc

N, E, H = 10000, 320000, 128
NC, NS = 2, 16
CH = 80
EPW = E // (NC * NS)
CPW = EPW // CH
NPT = 624
NREM = N - NS * NPT
K = 5
SLC = E // K
EPS = SLC // (NC * NS)
CPS = EPS // CH

_MESH = plsc.VectorSubcoreMesh(core_axis_name="c", subcore_axis_name="s")


@functools.partial(
    pl.kernel,
    mesh=_MESH,
    out_type=jax.ShapeDtypeStruct((NC, N, H), jnp.float32),
    scratch_types=[
        pltpu.VMEM((CPW, CH), jnp.int32),
        pltpu.VMEM((CH, H), jnp.float32),
        pltpu.VMEM((CH, H), jnp.float32),
        pltpu.VMEM_SHARED((N, H), jnp.float32),
        pltpu.SemaphoreType.DMA,
        pltpu.SemaphoreType.DMA,
    ],
)
def _seg_sum_sc(e_hbm, end_hbm, zeros_hbm, out_hbm, idx_v, rows0_v, rows1_v,
                acc_sh, sem0, sem1):
    c = lax.axis_index("c")
    s = lax.axis_index("s")
    pltpu.sync_copy(zeros_hbm.at[pl.ds(s * NPT, NPT)],
                    acc_sh.at[pl.ds(s * NPT, NPT)])

    @pl.when(s == NS - 1)
    def _zero_tail():
        pltpu.sync_copy(zeros_hbm.at[pl.ds(NS * NPT, NREM)],
                        acc_sh.at[pl.ds(NS * NPT, NREM)])

    pltpu.sync_copy(end_hbm.at[c, s], idx_v)
    plsc.subcore_barrier()
    wbase = (c * NS + s) * EPW

    def _start_in(i, rows, sem):
        pltpu.async_copy(e_hbm.at[pl.ds(wbase + i * CH, CH)], rows, sem)

    def _finish(i, rows, sem):
        pltpu.make_async_copy(e_hbm.at[pl.ds(wbase + i * CH, CH)],
                              rows, sem).wait()
        pltpu.sync_copy(rows, acc_sh.at[idx_v.at[i]], add=True)

    _start_in(0, rows0_v, sem0)

    def body(i, carry):
        b = lax.rem(i, 2)

        @pl.when(i + 1 < CPW)
        def _prefetch():
            @pl.when(b == 0)
            def _():
                _start_in(i + 1, rows1_v, sem1)

            @pl.when(b == 1)
            def _():
                _start_in(i + 1, rows0_v, sem0)

        @pl.when(b == 0)
        def _():
            _finish(i, rows0_v, sem0)

        @pl.when(b == 1)
        def _():
            _finish(i, rows1_v, sem1)

        return carry

    lax.fori_loop(0, CPW, body, 0)
    plsc.subcore_barrier()
    pltpu.sync_copy(acc_sh.at[pl.ds(s * NPT, NPT)],
                    out_hbm.at[c, pl.ds(s * NPT, NPT)])

    @pl.when(s == NS - 1)
    def _write_tail():
        pltpu.sync_copy(acc_sh.at[pl.ds(NS * NPT, NREM)],
                        out_hbm.at[c, pl.ds(NS * NPT, NREM)])


@functools.partial(
    pl.kernel,
    mesh=_MESH,
    out_type=jax.ShapeDtypeStruct((SLC, H), jnp.float32),
    scratch_types=[
        pltpu.VMEM((CPS, CH), jnp.int32),
        pltpu.VMEM((CPS, CH), jnp.int32),
        pltpu.VMEM((CH, H), jnp.float32),
        pltpu.VMEM((CH, H), jnp.float32),
        pltpu.VMEM((CH, H), jnp.float32),
        pltpu.VMEM((CH, H), jnp.float32),
        pltpu.SemaphoreType.DMA,
        pltpu.SemaphoreType.DMA,
        pltpu.SemaphoreType.DMA,
        pltpu.SemaphoreType.DMA,
    ],
)
def _gather_sc(ga_hbm, gb_hbm, start_hbm, end_hbm, out_hbm,
               idxs_v, idxe_v, bufa0_v, bufb0_v, bufa1_v, bufb1_v,
               sa0, sb0, sa1, sb1):
    c = lax.axis_index("c")
    s = lax.axis_index("s")
    pltpu.sync_copy(start_hbm.at[c, s], idxs_v)
    pltpu.sync_copy(end_hbm.at[c, s], idxe_v)
    wbase = (c * NS + s) * EPS

    def _start_in(i, ba, bb, sa, sb):
        pltpu.async_copy(ga_hbm.at[idxs_v.at[i]], ba, sa)
        pltpu.async_copy(gb_hbm.at[idxe_v.at[i]], bb, sb)

    def _finish(i, ba, bb, sa, sb):
        pltpu.make_async_copy(ga_hbm.at[idxs_v.at[i]], ba, sa).wait()
        pltpu.make_async_copy(gb_hbm.at[idxe_v.at[i]], bb, sb).wait()

        def add_row(r, carry):
            for k in range(H // 16):
                sl = pl.ds(k * 16, 16)
                ba[r, sl] = ba[r, sl] + bb[r, sl]
            return carry

        lax.fori_loop(0, CH, add_row, 0)
        pltpu.sync_copy(ba, out_hbm.at[pl.ds(wbase + i * CH, CH)])

    _start_in(0, bufa0_v, bufb0_v, sa0, sb0)

    def body(i, carry):
        b = lax.rem(i, 2)

        @pl.when(i + 1 < CPS)
        def _prefetch():
            @pl.when(b == 0)
            def _():
                _start_in(i + 1, bufa1_v, bufb1_v, sa1, sb1)

            @pl.when(b == 1)
            def _():
                _start_in(i + 1, bufa0_v, bufb0_v, sa0, sb0)

        @pl.when(b == 0)
        def _():
            _finish(i, bufa0_v, bufb0_v, sa0, sb0)

        @pl.when(b == 1)
        def _():
            _finish(i, bufa1_v, bufb1_v, sa1, sb1)

        return carry

    lax.fori_loop(0, CPS, body, 0)


def _node_body(x_ref, m_ref, w1a_ref, w1b_ref, b1_ref, w2_ref, b2_ref,
               wa_ref, wb_ref, be_ref, xout_ref, ga_ref, gb_ref):
    xb = x_ref[...]
    m = m_ref[0] + m_ref[1]
    h = jnp.maximum(
        jnp.dot(xb, w1a_ref[...], preferred_element_type=jnp.float32)
        + jnp.dot(m, w1b_ref[...], preferred_element_type=jnp.float32)
        + b1_ref[...], 0.0)
    xo = jnp.dot(h, w2_ref[...], preferred_element_type=jnp.float32) \
        + b2_ref[...] + xb
    xout_ref[...] = xo
    ga_ref[...] = jnp.dot(xo, wa_ref[...],
                          preferred_element_type=jnp.float32) + be_ref[...]
    gb_ref[...] = jnp.dot(xo, wb_ref[...], preferred_element_type=jnp.float32)


def _node_tc(x, msgs, w1a, w1b, b1n, w2n, b2n, wea, web, b1e):
    BN = 1000
    grid = (N // BN,)
    row_spec = pl.BlockSpec((BN, H), lambda i: (i, 0))
    w_spec = pl.BlockSpec((H, H), lambda i: (0, 0))
    b_spec = pl.BlockSpec((1, H), lambda i: (0, 0))
    return pl.pallas_call(
        _node_body,
        grid=grid,
        in_specs=[
            row_spec,
            pl.BlockSpec((NC, BN, H), lambda i: (0, i, 0)),
            w_spec, w_spec, b_spec, w_spec, b_spec, w_spec, w_spec, b_spec,
        ],
        out_specs=[row_spec, row_spec, row_spec],
        out_shape=[jax.ShapeDtypeStruct((N, H), jnp.float32)] * 3,
    )(x, msgs, w1a, w1b, b1n, w2n, b2n, wea, web, b1e)


def _edge_body(gsum_ref, e_ref, wc_ref, w2_ref, b2_ref, out_ref):
    eb = e_ref[...]
    h = jnp.maximum(
        gsum_ref[...]
        + jnp.dot(eb, wc_ref[...], preferred_element_type=jnp.float32), 0.0)
    out_ref[...] = jnp.dot(h, w2_ref[...],
                           preferred_element_type=jnp.float32) \
        + b2_ref[...] + eb


def _edge_body_acc(gsum_ref, e_ref, wc_ref, w2_ref, b2_ref, prev_ref, out_ref):
    del prev_ref
    _edge_body(gsum_ref, e_ref, wc_ref, w2_ref, b2_ref, out_ref)


_BE = 2000
_BPS = SLC // _BE


def _edge_tc(k, gsum_k, e, wec, w2e, b2e, prev):
    loc_spec = pl.BlockSpec((_BE, H), lambda i: (i, 0))
    shift_spec = pl.BlockSpec((_BE, H), lambda i, _k=k: (i + _k * _BPS, 0))
    w_spec = pl.BlockSpec((H, H), lambda i: (0, 0))
    b_spec = pl.BlockSpec((1, H), lambda i: (0, 0))
    body = _edge_body if prev is None else _edge_body_acc
    in_specs = [loc_spec, shift_spec, w_spec, w_spec, b_spec]
    args = [gsum_k, e, wec, w2e, b2e]
    aliases = {}
    if prev is not None:
        in_specs.append(pl.BlockSpec(memory_space=pl.ANY))
        args.append(prev)
        aliases = {5: 0}
    return pl.pallas_call(
        body,
        grid=(_BPS,),
        in_specs=in_specs,
        out_specs=shift_spec,
        out_shape=jax.ShapeDtypeStruct((E, H), jnp.float32),
        input_output_aliases=aliases,
    )(*args)


def kernel(x, edge_index, e, W1n, b1n, W2n, b2n, W1e, b1e, W2e, b2e):
    end = edge_index[1].reshape(NC, NS, CPW, CH)
    start5 = edge_index[0].reshape(K, NC, NS, CPS, CH)
    end5 = edge_index[1].reshape(K, NC, NS, CPS, CH)
    zeros = jnp.zeros((N, H), jnp.float32)

    msgs = _seg_sum_sc(e, end, zeros)

    x_out, ga_nodes, gb_nodes = _node_tc(
        x, msgs,
        W1n[:H], W1n[H:], b1n.reshape(1, H), W2n, b2n.reshape(1, H),
        W1e[:H], W1e[H:2 * H], b1e.reshape(1, H))

    wec = W1e[2 * H:]
    b2e_r = b2e.reshape(1, H)
    gsums = [_gather_sc(ga_nodes, gb_nodes, start5[k], end5[k])
             for k in range(K)]
    e_out = None
    for k in range(K):
        e_out = _edge_tc(k, gsums[k], e, wec, W2e, b2e_r, e_out)
    return (x_out, e_out)
